# baseline (speedup 1.0000x reference)
import functools

import jax
import jax.numpy as jnp
from jax.experimental import pallas as pl
from jax.experimental.pallas import tpu as pltpu
from jax.experimental.pallas import tpu_sc as plsc

_N = 10000
_E = 320000
_G = 64
_DIN = 128
_H = 64
_LAT = 32
_PC = 8
_EPS = 1e-5

_NC = 2
_NS = 16
_NT = _NC * _NS
_CH = 128
_NCH = 80
_EPT = _CH * _NCH
_EPAD = _NT * _EPT
_NPAD = 10240
_RPT = _NPAD // _NS

_PREC = jax.lax.Precision.HIGHEST
_IBN = 0.9999950000374997


def _mm(a, b):
    dims = (((a.ndim - 1,), (0,)), ((), ()))
    return jax.lax.dot_general(a, b, dims, precision=_PREC,
                               preferred_element_type=jnp.float32)


def _sc_mesh():
    return plsc.VectorSubcoreMesh(core_axis_name="c", subcore_axis_name="s",
                                  num_cores=_NC, num_subcores=_NS)


def _sc_deg(dstb):
    def body(dst_hbm, out_hbm, dst_v, ones_v, zer_v, acc_sh):
        c = jax.lax.axis_index("c")
        s = jax.lax.axis_index("s")
        w = c * _NS + s
        pltpu.sync_copy(dst_hbm.at[w], dst_v)
        zv = jnp.zeros((16,), jnp.float32)
        ov = jnp.ones((16,), jnp.float32)

        def fz(i, _):
            zer_v[pl.ds(i * 16, 16)] = zv
            return 0
        jax.lax.fori_loop(0, _RPT // 16, fz, 0)

        def fo(i, _):
            ones_v[pl.ds(i * 16, 16)] = ov
            return 0
        jax.lax.fori_loop(0, _CH // 16, fo, 0)

        pltpu.sync_copy(zer_v, acc_sh.at[pl.ds(s * _RPT, _RPT)])
        plsc.subcore_barrier()

        def eb(j, _):
            pltpu.sync_copy(ones_v, acc_sh.at[dst_v.at[j]], add=True)
            return 0
        jax.lax.fori_loop(0, _NCH, eb, 0)
        plsc.subcore_barrier()
        pltpu.sync_copy(acc_sh.at[pl.ds(s * _RPT, _RPT)],
                        out_hbm.at[c, pl.ds(s * _RPT, _RPT)])

    return pl.kernel(
        body,
        out_type=jax.ShapeDtypeStruct((_NC, _NPAD), jnp.float32),
        mesh=_sc_mesh(),
        scratch_types=[
            pltpu.VMEM((_NCH, _CH), jnp.int32),
            pltpu.VMEM((_CH,), jnp.float32),
            pltpu.VMEM((_RPT,), jnp.float32),
            pltpu.VMEM_SHARED((_NPAD,), jnp.float32),
        ],
    )(dstb)


def _sc_scatter(hp, srcb, dstb):
    def body(hp_hbm, src_hbm, dst_hbm, out_hbm,
             src_v, dst_v, rows_v, zer_v, acc_sh, gsem, ssem):
        c = jax.lax.axis_index("c")
        s = jax.lax.axis_index("s")
        w = c * _NS + s
        pltpu.sync_copy(src_hbm.at[w], src_v)
        pltpu.sync_copy(dst_hbm.at[w], dst_v)
        zv = jnp.zeros((16,), jnp.float32)

        def fz(i, _):
            zer_v[i // 4, pl.ds((i % 4) * 16, 16)] = zv
            return 0
        jax.lax.fori_loop(0, 64 * (_H // 16), fz, 0)

        def zc(i, _):
            pltpu.sync_copy(zer_v, acc_sh.at[pl.ds(s * _RPT + i * 64, 64)])
            return 0
        jax.lax.fori_loop(0, _RPT // 64, zc, 0)
        plsc.subcore_barrier()

        for p in range(4):
            pltpu.async_copy(hp_hbm.at[src_v.at[p]], rows_v.at[p], gsem.at[p])

        def eb(i, _):
            for b in range(8):
                jj = 8 * i + b

                @pl.when(jj >= 4)
                def _():
                    pltpu.make_async_copy(
                        rows_v.at[(b + 4) % 8],
                        acc_sh.at[dst_v.at[jj - 4]],
                        ssem.at[(b + 4) % 8]).wait()

                @pl.when(jj + 4 < _NCH)
                def _():
                    pltpu.async_copy(hp_hbm.at[src_v.at[jj + 4]],
                                     rows_v.at[(b + 4) % 8],
                                     gsem.at[(b + 4) % 8])

                pltpu.make_async_copy(hp_hbm.at[src_v.at[jj]],
                                      rows_v.at[b], gsem.at[b]).wait()
                pltpu.async_copy(rows_v.at[b], acc_sh.at[dst_v.at[jj]],
                                 ssem.at[b], add=True)
            return 0
        jax.lax.fori_loop(0, _NCH // 8, eb, 0)
        for t in range(4):
            b = (_NCH - 4 + t) % 8
            pltpu.make_async_copy(rows_v.at[b],
                                  acc_sh.at[dst_v.at[_NCH - 4 + t]],
                                  ssem.at[b]).wait()
        plsc.subcore_barrier()
        pltpu.sync_copy(acc_sh.at[pl.ds(s * _RPT, _RPT)],
                        out_hbm.at[c, pl.ds(s * _RPT, _RPT)])

    return pl.kernel(
        body,
        out_type=jax.ShapeDtypeStruct((_NC, _NPAD, _H), jnp.float32),
        mesh=_sc_mesh(),
        scratch_types=[
            pltpu.VMEM((_NCH, _CH), jnp.int32),
            pltpu.VMEM((_NCH, _CH), jnp.int32),
            pltpu.VMEM((8, _CH, _H), jnp.float32),
            pltpu.VMEM((64, _H), jnp.float32),
            pltpu.VMEM_SHARED((_NPAD, _H), jnp.float32),
            pltpu.SemaphoreType.DMA((8,)),
            pltpu.SemaphoreType.DMA((8,)),
        ],
        compiler_params=pltpu.CompilerParams(use_tc_tiling_on_sc=False),
    )(hp, srcb, dstb)


_BN = 2000
_NB = _N // _BN


def _row_spec(d):
    return pl.BlockSpec((_BN, d), lambda i: (i, 0))


def _whole_spec(shape):
    return pl.BlockSpec(shape, lambda i: tuple(0 for _ in shape))


def _tc_pre(x, W1, d0, d1):
    def body(x_r, w_r, d0_r, d1_r, hp_r, dv_r):
        dinv = jax.lax.rsqrt(d0_r[...] + d1_r[...] + 1.0)
        hp_r[...] = _mm(x_r[...], w_r[...]) * dinv
        dv_r[...] = dinv

    return pl.pallas_call(
        body,
        grid=(_NB,),
        in_specs=[_row_spec(_DIN), _whole_spec((_DIN, _H)),
                  _row_spec(1), _row_spec(1)],
        out_specs=[_row_spec(_H), _row_spec(1)],
        out_shape=[jax.ShapeDtypeStruct((_N, _H), jnp.float32),
                   jax.ShapeDtypeStruct((_N, 1), jnp.float32)],
    )(x, W1, d0, d1)


def _tc_mid(a0, a1, hp, dinv, b, Wn, xres):
    has_res = xres is not None

    def body(*refs):
        if has_res:
            a0_r, a1_r, hp_r, dv_r, b_r, w_r, xr_r, x_r, hpn_r = refs
        else:
            a0_r, a1_r, hp_r, dv_r, b_r, w_r, x_r, hpn_r = refs
        dinv = dv_r[...]
        hpv = hp_r[...]
        t = jnp.maximum(dinv * (a0_r[...] + a1_r[...] + hpv) + b_r[...], 0.0)
        if has_res:
            t = t + xr_r[...]
        x_r[...] = t
        hpn_r[...] = _mm(t, w_r[...]) * dinv

    args = (a0, a1, hp, dinv, b, Wn) + ((xres,) if has_res else ())
    in_specs = [_row_spec(_H), _row_spec(_H), _row_spec(_H), _row_spec(1),
                _whole_spec((1, _H)), _whole_spec((_H, _H))]
    if has_res:
        in_specs.append(_row_spec(_H))
    return pl.pallas_call(
        body,
        grid=(_NB,),
        in_specs=in_specs,
        out_specs=[_row_spec(_H), _row_spec(_H)],
        out_shape=[jax.ShapeDtypeStruct((_N, _H), jnp.float32),
                   jax.ShapeDtypeStruct((_N, _H), jnp.float32)],
    )(*args)


def _tc_final(a0, a1, hp, dinv, b4, xres, batch_col, physchem,
              ln_g, ln_b, pe_W1, pe_b1, pe_W2, pe_b2,
              proj_W1, proj_b1, proj_W2, proj_b2, gate_W, gate_b):
    neg = float("-inf")

    def body(a0_r, a1_r, hp_r, dv_r, b4_r, xr_r, bat_r, pc_r,
             lng_r, lnb_r, pw1_r, pb1_r, pw2_r, pb2_r,
             qw1_r, qb1_r, qw2_r, qb2_r, gw_r, gb_r, z_r,
             psum_s, cnt_s, pmax_s):
        i = pl.program_id(0)

        @pl.when(i == 0)
        def _():
            psum_s[...] = jnp.zeros((_G, _H), jnp.float32)
            cnt_s[...] = jnp.zeros((_G, 1), jnp.float32)
            pmax_s[...] = jnp.full((_G, _H), neg, jnp.float32)

        dinv = dv_r[...]
        x4 = jnp.maximum(dinv * (a0_r[...] + a1_r[...] + hp_r[...]) + b4_r[...],
                         0.0) + xr_r[...]
        batv = bat_r[...]
        gio = jax.lax.broadcasted_iota(jnp.int32, (_BN, _G), 1)
        maskf = (batv == gio).astype(jnp.float32)
        psum_s[...] += jax.lax.dot_general(
            maskf, x4, (((0,), (0,)), ((), ())), precision=_PREC,
            preferred_element_type=jnp.float32)
        cnt_s[...] += jax.lax.dot_general(
            maskf, jnp.ones((_BN, 1), jnp.float32), (((0,), (0,)), ((), ())),
            precision=_PREC, preferred_element_type=jnp.float32)
        rio = jax.lax.broadcasted_iota(jnp.int32, (_G, _H), 0)

        def gbody(g, pm):
            mg = batv == g
            m = jnp.max(jnp.where(mg, x4, neg), axis=0)
            return jnp.where(rio == g, jnp.maximum(pm, m[None, :]), pm)

        pmax_s[...] = jax.lax.fori_loop(0, _G, gbody, pmax_s[...])

        @pl.when(i == _NB - 1)
        def _():
            pool_sum = psum_s[...]
            cnt = cnt_s[...]
            pool_mean = pool_sum / jnp.maximum(cnt, 1.0)
            pooled = jnp.concatenate([pool_mean, pmax_s[...], pool_sum], axis=1)

            p = pc_r[...]
            mu = jnp.mean(p, axis=1, keepdims=True)
            var = jnp.mean((p - mu) ** 2, axis=1, keepdims=True)
            pcn = (p - mu) * jax.lax.rsqrt(var + _EPS) * lng_r[...] + lnb_r[...]
            pe = jnp.maximum(_mm(pcn, pw1_r[...]) + pb1_r[...], 0.0)
            pe = jnp.maximum(_mm(pe, pw2_r[...]) + pb2_r[...], 0.0)

            comb = jnp.concatenate([pooled, pe], axis=1)
            gate = jax.nn.sigmoid(_mm(comb, gw_r[...]) + gb_r[...])
            cf = jnp.concatenate([gate * pooled, (1.0 - gate) * pe], axis=1)
            h = jnp.maximum((_mm(cf, qw1_r[...]) + qb1_r[...]) * _IBN, 0.0)
            z_r[...] = (_mm(h, qw2_r[...]) + qb2_r[...]) * _IBN

    return pl.pallas_call(
        body,
        grid=(_NB,),
        in_specs=[_row_spec(_H), _row_spec(_H), _row_spec(_H), _row_spec(1),
                  _whole_spec((1, _H)), _row_spec(_H), _row_spec(1),
                  _whole_spec((_G, _PC)),
                  _whole_spec((1, _PC)), _whole_spec((1, _PC)),
                  _whole_spec((_PC, _H)), _whole_spec((1, _H)),
                  _whole_spec((_H, _H)), _whole_spec((1, _H)),
                  _whole_spec((4 * _H, 2 * _H)), _whole_spec((1, 2 * _H)),
                  _whole_spec((2 * _H, _LAT)), _whole_spec((1, _LAT)),
                  _whole_spec((4 * _H, 1)), _whole_spec((1, 1))],
        out_specs=_whole_spec((_G, _LAT)),
        out_shape=jax.ShapeDtypeStruct((_G, _LAT), jnp.float32),
        scratch_shapes=[pltpu.VMEM((_G, _H), jnp.float32),
                        pltpu.VMEM((_G, 1), jnp.float32),
                        pltpu.VMEM((_G, _H), jnp.float32)],
    )(a0, a1, hp, dinv, b4, xres, batch_col, physchem,
      ln_g, ln_b, pe_W1, pe_b1, pe_W2, pe_b2,
      proj_W1, proj_b1, proj_W2, proj_b2, gate_W, gate_b)


def kernel(x, edge_index, batch, physchem, W1, b1, W2, b2, W3, b3, W4, b4,
           ln_g, ln_b, pe_W1, pe_b1, pe_W2, pe_b2,
           proj_W1, proj_b1, proj_W2, proj_b2, gate_W, gate_b):
    src = edge_index[0]
    dst = edge_index[1]
    pad = _EPAD - _E
    srcb = jnp.concatenate([src, jnp.zeros((pad,), jnp.int32)]
                           ).reshape(_NT, _NCH, _CH)
    dstb = jnp.concatenate([dst, jnp.full((pad,), _N, jnp.int32)]
                           ).reshape(_NT, _NCH, _CH)

    deg2 = _sc_deg(dstb)
    d0 = deg2[0, :_N, None]
    d1 = deg2[1, :_N, None]

    hp1, dinv = _tc_pre(x, W1, d0, d1)

    acc = _sc_scatter(hp1, srcb, dstb)
    x1, hp2 = _tc_mid(acc[0, :_N], acc[1, :_N], hp1, dinv,
                      b1[None, :], W2, None)
    acc = _sc_scatter(hp2, srcb, dstb)
    x2, hp3 = _tc_mid(acc[0, :_N], acc[1, :_N], hp2, dinv,
                      b2[None, :], W3, x1)
    acc = _sc_scatter(hp3, srcb, dstb)
    x3, hp4 = _tc_mid(acc[0, :_N], acc[1, :_N], hp3, dinv,
                      b3[None, :], W4, x2)
    acc = _sc_scatter(hp4, srcb, dstb)

    return _tc_final(acc[0, :_N], acc[1, :_N], hp4, dinv, b4[None, :], x3,
                     batch[:, None], physchem,
                     ln_g[None, :], ln_b[None, :], pe_W1, pe_b1[None, :],
                     pe_W2, pe_b2[None, :], proj_W1, proj_b1[None, :],
                     proj_W2, proj_b2[None, :], gate_W, gate_b[None, :])


# speedup vs baseline: 1.2410x; 1.2410x over previous
import functools

import jax
import jax.numpy as jnp
from jax.experimental import pallas as pl
from jax.experimental.pallas import tpu as pltpu
from jax.experimental.pallas import tpu_sc as plsc

_N = 10000
_E = 320000
_G = 64
_DIN = 128
_H = 64
_LAT = 32
_PC = 8
_EPS = 1e-5

_NC = 2
_NS = 16
_NT = _NC * _NS
_CH = 128
_NCH0 = 136
_NCH1 = 24
_EPAD = _NS * _CH * (_NCH0 + _NCH1)
_NPAD = 10240
_RPT = _NPAD // _NS

_PREC = jax.lax.Precision.HIGHEST
_IBN = 0.9999950000374997


def _mm(a, b):
    dims = (((a.ndim - 1,), (0,)), ((), ()))
    return jax.lax.dot_general(a, b, dims, precision=_PREC,
                               preferred_element_type=jnp.float32)


def _sc_mesh():
    return plsc.VectorSubcoreMesh(core_axis_name="c", subcore_axis_name="s",
                                  num_cores=_NC, num_subcores=_NS)


def _sc_deg(dstb):
    def body(dst_hbm, out_hbm, dst_v, ones_v, zer_v, acc_sh):
        c = jax.lax.axis_index("c")
        s = jax.lax.axis_index("s")
        w = c * _NS + s
        nch = jnp.where(c == 0, _NCH0, _NCH1)
        pltpu.sync_copy(dst_hbm.at[w], dst_v)
        zv = jnp.zeros((16,), jnp.float32)
        ov = jnp.ones((16,), jnp.float32)

        def fz(i, _):
            zer_v[pl.ds(i * 16, 16)] = zv
            return 0
        jax.lax.fori_loop(0, _RPT // 16, fz, 0)

        def fo(i, _):
            ones_v[pl.ds(i * 16, 16)] = ov
            return 0
        jax.lax.fori_loop(0, _CH // 16, fo, 0)

        pltpu.sync_copy(zer_v, acc_sh.at[pl.ds(s * _RPT, _RPT)])
        plsc.subcore_barrier()

        def eb(j, _):
            pltpu.sync_copy(ones_v, acc_sh.at[dst_v.at[j]], add=True)
            return 0
        jax.lax.fori_loop(0, nch, eb, 0)
        plsc.subcore_barrier()
        pltpu.sync_copy(acc_sh.at[pl.ds(s * _RPT, _RPT)],
                        out_hbm.at[c, pl.ds(s * _RPT, _RPT)])

    return pl.kernel(
        body,
        out_type=jax.ShapeDtypeStruct((_NC, _NPAD), jnp.float32),
        mesh=_sc_mesh(),
        scratch_types=[
            pltpu.VMEM((_NCH0, _CH), jnp.int32),
            pltpu.VMEM((_CH,), jnp.float32),
            pltpu.VMEM((_RPT,), jnp.float32),
            pltpu.VMEM_SHARED((_NPAD,), jnp.float32),
        ],
    )(dstb)


def _sc_scatter(hp, srcb, dstb):
    def body(hp_hbm, src_hbm, dst_hbm, out_hbm,
             src_v, dst_v, rows_v, zer_v, acc_sh, gsem, ssem):
        c = jax.lax.axis_index("c")
        s = jax.lax.axis_index("s")
        w = c * _NS + s
        nch = jnp.where(c == 0, _NCH0, _NCH1)
        pltpu.sync_copy(src_hbm.at[w], src_v)
        pltpu.sync_copy(dst_hbm.at[w], dst_v)
        zv = jnp.zeros((16,), jnp.float32)

        def fz(i, _):
            zer_v[i // 4, pl.ds((i % 4) * 16, 16)] = zv
            return 0
        jax.lax.fori_loop(0, 64 * (_H // 16), fz, 0)

        def zc(i, _):
            pltpu.sync_copy(zer_v, acc_sh.at[pl.ds(s * _RPT + i * 64, 64)])
            return 0
        jax.lax.fori_loop(0, _RPT // 64, zc, 0)
        plsc.subcore_barrier()

        for p in range(2):
            pltpu.async_copy(hp_hbm.at[src_v.at[p]], rows_v.at[p], gsem.at[p])

        def eb(i, _):
            for b in range(4):
                jj = 4 * i + b

                @pl.when(jj >= 2)
                def _():
                    pltpu.make_async_copy(
                        rows_v.at[(b + 2) % 4],
                        acc_sh.at[dst_v.at[jj - 2]],
                        ssem.at[(b + 2) % 4]).wait()

                @pl.when(jj + 2 < nch)
                def _():
                    pltpu.async_copy(hp_hbm.at[src_v.at[jj + 2]],
                                     rows_v.at[(b + 2) % 4],
                                     gsem.at[(b + 2) % 4])

                pltpu.make_async_copy(hp_hbm.at[src_v.at[jj]],
                                      rows_v.at[b], gsem.at[b]).wait()
                pltpu.async_copy(rows_v.at[b], acc_sh.at[dst_v.at[jj]],
                                 ssem.at[b], add=True)
            return 0
        jax.lax.fori_loop(0, nch // 4, eb, 0)
        for t in range(2):
            b = 2 + t
            pltpu.make_async_copy(rows_v.at[b],
                                  acc_sh.at[dst_v.at[nch - 2 + t]],
                                  ssem.at[b]).wait()
        plsc.subcore_barrier()
        pltpu.sync_copy(acc_sh.at[pl.ds(s * _RPT, _RPT)],
                        out_hbm.at[c, pl.ds(s * _RPT, _RPT)])

    return pl.kernel(
        body,
        out_type=jax.ShapeDtypeStruct((_NC, _NPAD, _H), jnp.float32),
        mesh=_sc_mesh(),
        scratch_types=[
            pltpu.VMEM((_NCH0, _CH), jnp.int32),
            pltpu.VMEM((_NCH0, _CH), jnp.int32),
            pltpu.VMEM((4, _CH, _H), jnp.float32),
            pltpu.VMEM((64, _H), jnp.float32),
            pltpu.VMEM_SHARED((_NPAD, _H), jnp.float32),
            pltpu.SemaphoreType.DMA((4,)),
            pltpu.SemaphoreType.DMA((4,)),
        ],
        compiler_params=pltpu.CompilerParams(use_tc_tiling_on_sc=False),
    )(hp, srcb, dstb)


_BN = 2000
_NB = _N // _BN


def _row_spec(d):
    return pl.BlockSpec((_BN, d), lambda i: (i, 0))


def _whole_spec(shape):
    return pl.BlockSpec(shape, lambda i: tuple(0 for _ in shape))


def _tc_pre(x, W1, d0, d1):
    def body(x_r, w_r, d0_r, d1_r, hp_r, dv_r):
        dinv = jax.lax.rsqrt(d0_r[...] + d1_r[...] + 1.0)
        hp_r[...] = _mm(x_r[...], w_r[...]) * dinv
        dv_r[...] = dinv

    return pl.pallas_call(
        body,
        grid=(_NB,),
        in_specs=[_row_spec(_DIN), _whole_spec((_DIN, _H)),
                  _row_spec(1), _row_spec(1)],
        out_specs=[_row_spec(_H), _row_spec(1)],
        out_shape=[jax.ShapeDtypeStruct((_N, _H), jnp.float32),
                   jax.ShapeDtypeStruct((_N, 1), jnp.float32)],
    )(x, W1, d0, d1)


def _tc_mid(a0, a1, hp, dinv, b, Wn, xres):
    has_res = xres is not None

    def body(*refs):
        if has_res:
            a0_r, a1_r, hp_r, dv_r, b_r, w_r, xr_r, x_r, hpn_r = refs
        else:
            a0_r, a1_r, hp_r, dv_r, b_r, w_r, x_r, hpn_r = refs
        dinv = dv_r[...]
        hpv = hp_r[...]
        t = jnp.maximum(dinv * (a0_r[...] + a1_r[...] + hpv) + b_r[...], 0.0)
        if has_res:
            t = t + xr_r[...]
        x_r[...] = t
        hpn_r[...] = _mm(t, w_r[...]) * dinv

    args = (a0, a1, hp, dinv, b, Wn) + ((xres,) if has_res else ())
    in_specs = [_row_spec(_H), _row_spec(_H), _row_spec(_H), _row_spec(1),
                _whole_spec((1, _H)), _whole_spec((_H, _H))]
    if has_res:
        in_specs.append(_row_spec(_H))
    return pl.pallas_call(
        body,
        grid=(_NB,),
        in_specs=in_specs,
        out_specs=[_row_spec(_H), _row_spec(_H)],
        out_shape=[jax.ShapeDtypeStruct((_N, _H), jnp.float32),
                   jax.ShapeDtypeStruct((_N, _H), jnp.float32)],
    )(*args)


def _tc_final(a0, a1, hp, dinv, b4, xres, batch_col, physchem,
              ln_g, ln_b, pe_W1, pe_b1, pe_W2, pe_b2,
              proj_W1, proj_b1, proj_W2, proj_b2, gate_W, gate_b):
    neg = float("-inf")

    def body(a0_r, a1_r, hp_r, dv_r, b4_r, xr_r, bat_r, pc_r,
             lng_r, lnb_r, pw1_r, pb1_r, pw2_r, pb2_r,
             qw1_r, qb1_r, qw2_r, qb2_r, gw_r, gb_r, z_r,
             psum_s, cnt_s, pmax_s):
        i = pl.program_id(0)

        @pl.when(i == 0)
        def _():
            psum_s[...] = jnp.zeros((_G, _H), jnp.float32)
            cnt_s[...] = jnp.zeros((_G, 1), jnp.float32)
            pmax_s[...] = jnp.full((_G, _H), neg, jnp.float32)

        dinv = dv_r[...]
        x4 = jnp.maximum(dinv * (a0_r[...] + a1_r[...] + hp_r[...]) + b4_r[...],
                         0.0) + xr_r[...]
        batv = bat_r[...]
        gio = jax.lax.broadcasted_iota(jnp.int32, (_BN, _G), 1)
        maskf = (batv == gio).astype(jnp.float32)
        psum_s[...] += jax.lax.dot_general(
            maskf, x4, (((0,), (0,)), ((), ())), precision=_PREC,
            preferred_element_type=jnp.float32)
        cnt_s[...] += jax.lax.dot_general(
            maskf, jnp.ones((_BN, 1), jnp.float32), (((0,), (0,)), ((), ())),
            precision=_PREC, preferred_element_type=jnp.float32)
        rio = jax.lax.broadcasted_iota(jnp.int32, (_G, _H), 0)

        def gbody(g, pm):
            mg = batv == g
            m = jnp.max(jnp.where(mg, x4, neg), axis=0)
            return jnp.where(rio == g, jnp.maximum(pm, m[None, :]), pm)

        pmax_s[...] = jax.lax.fori_loop(0, _G, gbody, pmax_s[...])

        @pl.when(i == _NB - 1)
        def _():
            pool_sum = psum_s[...]
            cnt = cnt_s[...]
            pool_mean = pool_sum / jnp.maximum(cnt, 1.0)
            pooled = jnp.concatenate([pool_mean, pmax_s[...], pool_sum], axis=1)

            p = pc_r[...]
            mu = jnp.mean(p, axis=1, keepdims=True)
            var = jnp.mean((p - mu) ** 2, axis=1, keepdims=True)
            pcn = (p - mu) * jax.lax.rsqrt(var + _EPS) * lng_r[...] + lnb_r[...]
            pe = jnp.maximum(_mm(pcn, pw1_r[...]) + pb1_r[...], 0.0)
            pe = jnp.maximum(_mm(pe, pw2_r[...]) + pb2_r[...], 0.0)

            comb = jnp.concatenate([pooled, pe], axis=1)
            gate = jax.nn.sigmoid(_mm(comb, gw_r[...]) + gb_r[...])
            cf = jnp.concatenate([gate * pooled, (1.0 - gate) * pe], axis=1)
            h = jnp.maximum((_mm(cf, qw1_r[...]) + qb1_r[...]) * _IBN, 0.0)
            z_r[...] = (_mm(h, qw2_r[...]) + qb2_r[...]) * _IBN

    return pl.pallas_call(
        body,
        grid=(_NB,),
        in_specs=[_row_spec(_H), _row_spec(_H), _row_spec(_H), _row_spec(1),
                  _whole_spec((1, _H)), _row_spec(_H), _row_spec(1),
                  _whole_spec((_G, _PC)),
                  _whole_spec((1, _PC)), _whole_spec((1, _PC)),
                  _whole_spec((_PC, _H)), _whole_spec((1, _H)),
                  _whole_spec((_H, _H)), _whole_spec((1, _H)),
                  _whole_spec((4 * _H, 2 * _H)), _whole_spec((1, 2 * _H)),
                  _whole_spec((2 * _H, _LAT)), _whole_spec((1, _LAT)),
                  _whole_spec((4 * _H, 1)), _whole_spec((1, 1))],
        out_specs=_whole_spec((_G, _LAT)),
        out_shape=jax.ShapeDtypeStruct((_G, _LAT), jnp.float32),
        scratch_shapes=[pltpu.VMEM((_G, _H), jnp.float32),
                        pltpu.VMEM((_G, 1), jnp.float32),
                        pltpu.VMEM((_G, _H), jnp.float32)],
    )(a0, a1, hp, dinv, b4, xres, batch_col, physchem,
      ln_g, ln_b, pe_W1, pe_b1, pe_W2, pe_b2,
      proj_W1, proj_b1, proj_W2, proj_b2, gate_W, gate_b)


def kernel(x, edge_index, batch, physchem, W1, b1, W2, b2, W3, b3, W4, b4,
           ln_g, ln_b, pe_W1, pe_b1, pe_W2, pe_b2,
           proj_W1, proj_b1, proj_W2, proj_b2, gate_W, gate_b):
    src = edge_index[0]
    dst = edge_index[1]
    pad = _EPAD - _E
    tot0 = _NS * _NCH0 * _CH

    def _blocks(flat):
        p0 = flat[:tot0].reshape(_NS, _NCH0, _CH)
        p1 = flat[tot0:].reshape(_NS, _NCH1, _CH)
        p1 = jnp.pad(p1, ((0, 0), (0, _NCH0 - _NCH1), (0, 0)))
        return jnp.concatenate([p0, p1], axis=0)

    srcb = _blocks(jnp.concatenate([src, jnp.zeros((pad,), jnp.int32)]))
    dstb = _blocks(jnp.concatenate([dst, jnp.full((pad,), _N, jnp.int32)]))

    deg2 = _sc_deg(dstb)
    d0 = deg2[0, :_N, None]
    d1 = deg2[1, :_N, None]

    hp1, dinv = _tc_pre(x, W1, d0, d1)

    acc = _sc_scatter(hp1, srcb, dstb)
    x1, hp2 = _tc_mid(acc[0, :_N], acc[1, :_N], hp1, dinv,
                      b1[None, :], W2, None)
    acc = _sc_scatter(hp2, srcb, dstb)
    x2, hp3 = _tc_mid(acc[0, :_N], acc[1, :_N], hp2, dinv,
                      b2[None, :], W3, x1)
    acc = _sc_scatter(hp3, srcb, dstb)
    x3, hp4 = _tc_mid(acc[0, :_N], acc[1, :_N], hp3, dinv,
                      b3[None, :], W4, x2)
    acc = _sc_scatter(hp4, srcb, dstb)

    return _tc_final(acc[0, :_N], acc[1, :_N], hp4, dinv, b4[None, :], x3,
                     batch[:, None], physchem,
                     ln_g[None, :], ln_b[None, :], pe_W1, pe_b1[None, :],
                     pe_W2, pe_b2[None, :], proj_W1, proj_b1[None, :],
                     proj_W2, proj_b2[None, :], gate_W, gate_b[None, :])


# speedup vs baseline: 1.3845x; 1.1157x over previous
import functools

import jax
import jax.numpy as jnp
from jax.experimental import pallas as pl
from jax.experimental.pallas import tpu as pltpu
from jax.experimental.pallas import tpu_sc as plsc

_N = 10000
_E = 320000
_G = 64
_DIN = 128
_H = 64
_LAT = 32
_PC = 8
_EPS = 1e-5

_NC = 2
_NS = 16
_NT = _NC * _NS
_CH = 128
_NCH0 = 152
_NCH1 = 8
_EPAD = _NS * _CH * (_NCH0 + _NCH1)
_NPAD = 10240
_RPT = _NPAD // _NS

_PREC = jax.lax.Precision.HIGHEST
_IBN = 0.9999950000374997


def _mm(a, b):
    dims = (((a.ndim - 1,), (0,)), ((), ()))
    return jax.lax.dot_general(a, b, dims, precision=_PREC,
                               preferred_element_type=jnp.float32)


def _sc_mesh():
    return plsc.VectorSubcoreMesh(core_axis_name="c", subcore_axis_name="s",
                                  num_cores=_NC, num_subcores=_NS)


def _sc_deg(dstb):
    def body(dst_hbm, out_hbm, dst_v, ones_v, zer_v, acc_sh):
        c = jax.lax.axis_index("c")
        s = jax.lax.axis_index("s")
        w = c * _NS + s
        nch = jnp.where(c == 0, _NCH0, _NCH1)
        pltpu.sync_copy(dst_hbm.at[w], dst_v)
        zv = jnp.zeros((16,), jnp.float32)
        ov = jnp.ones((16,), jnp.float32)

        def fz(i, _):
            zer_v[pl.ds(i * 16, 16)] = zv
            return 0
        jax.lax.fori_loop(0, _RPT // 16, fz, 0)

        def fo(i, _):
            ones_v[pl.ds(i * 16, 16)] = ov
            return 0
        jax.lax.fori_loop(0, _CH // 16, fo, 0)

        pltpu.sync_copy(zer_v, acc_sh.at[pl.ds(s * _RPT, _RPT)])
        plsc.subcore_barrier()

        def eb(j, _):
            pltpu.sync_copy(ones_v, acc_sh.at[dst_v.at[j]], add=True)
            return 0
        jax.lax.fori_loop(0, nch, eb, 0)
        plsc.subcore_barrier()
        pltpu.sync_copy(acc_sh.at[pl.ds(s * _RPT, _RPT)],
                        out_hbm.at[c, pl.ds(s * _RPT, _RPT)])

    return pl.kernel(
        body,
        out_type=jax.ShapeDtypeStruct((_NC, _NPAD), jnp.float32),
        mesh=_sc_mesh(),
        scratch_types=[
            pltpu.VMEM((_NCH0, _CH), jnp.int32),
            pltpu.VMEM((_CH,), jnp.float32),
            pltpu.VMEM((_RPT,), jnp.float32),
            pltpu.VMEM_SHARED((_NPAD,), jnp.float32),
        ],
    )(dstb)


def _sc_scatter(hp, srcb, dstb):
    def body(hp_hbm, src_hbm, dst_hbm, out_hbm,
             src_v, dst_v, rows_v, zer_v, acc_sh, gsem, ssem):
        c = jax.lax.axis_index("c")
        s = jax.lax.axis_index("s")
        w = c * _NS + s
        nch = jnp.where(c == 0, _NCH0, _NCH1)
        pltpu.sync_copy(src_hbm.at[w], src_v)
        pltpu.sync_copy(dst_hbm.at[w], dst_v)
        zv = jnp.zeros((16,), jnp.float32)

        def fz(i, _):
            zer_v[i // 4, pl.ds((i % 4) * 16, 16)] = zv
            return 0
        jax.lax.fori_loop(0, 64 * (_H // 16), fz, 0)

        def zc(i, _):
            pltpu.sync_copy(zer_v, acc_sh.at[pl.ds(s * _RPT + i * 64, 64)])
            return 0
        jax.lax.fori_loop(0, _RPT // 64, zc, 0)
        plsc.subcore_barrier()

        for p in range(2):
            pltpu.async_copy(hp_hbm.at[src_v.at[p]], rows_v.at[p], gsem.at[p])

        def eb(i, _):
            for b in range(4):
                jj = 4 * i + b

                @pl.when(jj >= 2)
                def _():
                    pltpu.make_async_copy(
                        rows_v.at[(b + 2) % 4],
                        acc_sh.at[dst_v.at[jj - 2]],
                        ssem.at[(b + 2) % 4]).wait()

                @pl.when(jj + 2 < nch)
                def _():
                    pltpu.async_copy(hp_hbm.at[src_v.at[jj + 2]],
                                     rows_v.at[(b + 2) % 4],
                                     gsem.at[(b + 2) % 4])

                pltpu.make_async_copy(hp_hbm.at[src_v.at[jj]],
                                      rows_v.at[b], gsem.at[b]).wait()
                pltpu.async_copy(rows_v.at[b], acc_sh.at[dst_v.at[jj]],
                                 ssem.at[b], add=True)
            return 0
        jax.lax.fori_loop(0, nch // 4, eb, 0)
        for t in range(2):
            b = 2 + t
            pltpu.make_async_copy(rows_v.at[b],
                                  acc_sh.at[dst_v.at[nch - 2 + t]],
                                  ssem.at[b]).wait()
        plsc.subcore_barrier()
        pltpu.sync_copy(acc_sh.at[pl.ds(s * _RPT, _RPT)],
                        out_hbm.at[c, pl.ds(s * _RPT, _RPT)])

    return pl.kernel(
        body,
        out_type=jax.ShapeDtypeStruct((_NC, _NPAD, _H), jnp.float32),
        mesh=_sc_mesh(),
        scratch_types=[
            pltpu.VMEM((_NCH0, _CH), jnp.int32),
            pltpu.VMEM((_NCH0, _CH), jnp.int32),
            pltpu.VMEM((4, _CH, _H), jnp.float32),
            pltpu.VMEM((64, _H), jnp.float32),
            pltpu.VMEM_SHARED((_NPAD, _H), jnp.float32),
            pltpu.SemaphoreType.DMA((4,)),
            pltpu.SemaphoreType.DMA((4,)),
        ],
        compiler_params=pltpu.CompilerParams(use_tc_tiling_on_sc=False),
    )(hp, srcb, dstb)


_BN = 2000
_NB = _N // _BN


def _row_spec(d):
    return pl.BlockSpec((_BN, d), lambda i: (i, 0))


def _whole_spec(shape):
    return pl.BlockSpec(shape, lambda i: tuple(0 for _ in shape))


def _tc_pre(x, W1, d0, d1):
    def body(x_r, w_r, d0_r, d1_r, hp_r, dv_r):
        dinv = jax.lax.rsqrt(d0_r[...] + d1_r[...] + 1.0)
        hp_r[...] = _mm(x_r[...], w_r[...]) * dinv
        dv_r[...] = dinv

    return pl.pallas_call(
        body,
        grid=(_NB,),
        in_specs=[_row_spec(_DIN), _whole_spec((_DIN, _H)),
                  _row_spec(1), _row_spec(1)],
        out_specs=[_row_spec(_H), _row_spec(1)],
        out_shape=[jax.ShapeDtypeStruct((_N, _H), jnp.float32),
                   jax.ShapeDtypeStruct((_N, 1), jnp.float32)],
    )(x, W1, d0, d1)


_acc_spec = pl.BlockSpec((2, _BN, _H), lambda i: (0, i, 0))


def _tc_mid(acc, hp, dinv, b, Wn, xres):
    has_res = xres is not None

    def body(*refs):
        if has_res:
            a_r, hp_r, dv_r, b_r, w_r, xr_r, x_r, hpn_r = refs
        else:
            a_r, hp_r, dv_r, b_r, w_r, x_r, hpn_r = refs
        accb = a_r[...]
        dinv = dv_r[...]
        hpv = hp_r[...]
        t = jnp.maximum(dinv * (accb[0] + accb[1] + hpv) + b_r[...], 0.0)
        if has_res:
            t = t + xr_r[...]
        x_r[...] = t
        hpn_r[...] = _mm(t, w_r[...]) * dinv

    args = (acc, hp, dinv, b, Wn) + ((xres,) if has_res else ())
    in_specs = [_acc_spec, _row_spec(_H), _row_spec(1),
                _whole_spec((1, _H)), _whole_spec((_H, _H))]
    if has_res:
        in_specs.append(_row_spec(_H))
    return pl.pallas_call(
        body,
        grid=(_NB,),
        in_specs=in_specs,
        out_specs=[_row_spec(_H), _row_spec(_H)],
        out_shape=[jax.ShapeDtypeStruct((_N, _H), jnp.float32),
                   jax.ShapeDtypeStruct((_N, _H), jnp.float32)],
    )(*args)


def _tc_final(acc, hp, dinv, b4, xres, batch_col, physchem,
              ln_g, ln_b, pe_W1, pe_b1, pe_W2, pe_b2,
              proj_W1, proj_b1, proj_W2, proj_b2, gate_W, gate_b):
    neg = float("-inf")

    def body(a_r, hp_r, dv_r, b4_r, xr_r, bat_r, pc_r,
             lng_r, lnb_r, pw1_r, pb1_r, pw2_r, pb2_r,
             qw1_r, qb1_r, qw2_r, qb2_r, gw_r, gb_r, z_r,
             psum_s, cnt_s, pmax_s):
        i = pl.program_id(0)

        @pl.when(i == 0)
        def _():
            psum_s[...] = jnp.zeros((_G, _H), jnp.float32)
            cnt_s[...] = jnp.zeros((_G, 1), jnp.float32)
            pmax_s[...] = jnp.full((_G, _H), neg, jnp.float32)

        accb = a_r[...]
        dinv = dv_r[...]
        x4 = jnp.maximum(dinv * (accb[0] + accb[1] + hp_r[...]) + b4_r[...],
                         0.0) + xr_r[...]
        batv = bat_r[...]
        gio = jax.lax.broadcasted_iota(jnp.int32, (_BN, _G), 1)
        maskf = (batv == gio).astype(jnp.float32)
        psum_s[...] += jax.lax.dot_general(
            maskf, x4, (((0,), (0,)), ((), ())), precision=_PREC,
            preferred_element_type=jnp.float32)
        cnt_s[...] += jax.lax.dot_general(
            maskf, jnp.ones((_BN, 1), jnp.float32), (((0,), (0,)), ((), ())),
            precision=_PREC, preferred_element_type=jnp.float32)
        rio = jax.lax.broadcasted_iota(jnp.int32, (_G, _H), 0)
        gmin = bat_r[0, 0]
        gmax = bat_r[_BN - 1, 0]

        def gbody(g, _):
            @pl.when(jnp.logical_and(g >= gmin, g <= gmax))
            def _():
                mg = batv == g
                m = jnp.max(jnp.where(mg, x4, neg), axis=0)
                pmax_s[...] = jnp.where(rio == g,
                                        jnp.maximum(pmax_s[...], m[None, :]),
                                        pmax_s[...])
            return 0

        jax.lax.fori_loop(0, _G, gbody, 0)

        @pl.when(i == _NB - 1)
        def _():
            pool_sum = psum_s[...]
            cnt = cnt_s[...]
            pool_mean = pool_sum / jnp.maximum(cnt, 1.0)
            pooled = jnp.concatenate([pool_mean, pmax_s[...], pool_sum], axis=1)

            p = pc_r[...]
            mu = jnp.mean(p, axis=1, keepdims=True)
            var = jnp.mean((p - mu) ** 2, axis=1, keepdims=True)
            pcn = (p - mu) * jax.lax.rsqrt(var + _EPS) * lng_r[...] + lnb_r[...]
            pe = jnp.maximum(_mm(pcn, pw1_r[...]) + pb1_r[...], 0.0)
            pe = jnp.maximum(_mm(pe, pw2_r[...]) + pb2_r[...], 0.0)

            comb = jnp.concatenate([pooled, pe], axis=1)
            gate = jax.nn.sigmoid(_mm(comb, gw_r[...]) + gb_r[...])
            cf = jnp.concatenate([gate * pooled, (1.0 - gate) * pe], axis=1)
            h = jnp.maximum((_mm(cf, qw1_r[...]) + qb1_r[...]) * _IBN, 0.0)
            z_r[...] = (_mm(h, qw2_r[...]) + qb2_r[...]) * _IBN

    return pl.pallas_call(
        body,
        grid=(_NB,),
        in_specs=[_acc_spec, _row_spec(_H), _row_spec(1),
                  _whole_spec((1, _H)), _row_spec(_H), _row_spec(1),
                  _whole_spec((_G, _PC)),
                  _whole_spec((1, _PC)), _whole_spec((1, _PC)),
                  _whole_spec((_PC, _H)), _whole_spec((1, _H)),
                  _whole_spec((_H, _H)), _whole_spec((1, _H)),
                  _whole_spec((4 * _H, 2 * _H)), _whole_spec((1, 2 * _H)),
                  _whole_spec((2 * _H, _LAT)), _whole_spec((1, _LAT)),
                  _whole_spec((4 * _H, 1)), _whole_spec((1, 1))],
        out_specs=_whole_spec((_G, _LAT)),
        out_shape=jax.ShapeDtypeStruct((_G, _LAT), jnp.float32),
        scratch_shapes=[pltpu.VMEM((_G, _H), jnp.float32),
                        pltpu.VMEM((_G, 1), jnp.float32),
                        pltpu.VMEM((_G, _H), jnp.float32)],
    )(acc, hp, dinv, b4, xres, batch_col, physchem,
      ln_g, ln_b, pe_W1, pe_b1, pe_W2, pe_b2,
      proj_W1, proj_b1, proj_W2, proj_b2, gate_W, gate_b)


def kernel(x, edge_index, batch, physchem, W1, b1, W2, b2, W3, b3, W4, b4,
           ln_g, ln_b, pe_W1, pe_b1, pe_W2, pe_b2,
           proj_W1, proj_b1, proj_W2, proj_b2, gate_W, gate_b):
    src = edge_index[0]
    dst = edge_index[1]
    pad = _EPAD - _E
    tot0 = _NS * _NCH0 * _CH

    def _blocks(flat):
        p0 = flat[:tot0].reshape(_NS, _NCH0, _CH)
        p1 = flat[tot0:].reshape(_NS, _NCH1, _CH)
        p1 = jnp.pad(p1, ((0, 0), (0, _NCH0 - _NCH1), (0, 0)))
        return jnp.concatenate([p0, p1], axis=0)

    srcb = _blocks(jnp.concatenate([src, jnp.zeros((pad,), jnp.int32)]))
    dstb = _blocks(jnp.concatenate([dst, jnp.full((pad,), _N, jnp.int32)]))

    deg2 = _sc_deg(dstb)
    d0 = deg2[0, :_N, None]
    d1 = deg2[1, :_N, None]

    hp1, dinv = _tc_pre(x, W1, d0, d1)

    acc = _sc_scatter(hp1, srcb, dstb)
    x1, hp2 = _tc_mid(acc, hp1, dinv, b1[None, :], W2, None)
    acc = _sc_scatter(hp2, srcb, dstb)
    x2, hp3 = _tc_mid(acc, hp2, dinv, b2[None, :], W3, x1)
    acc = _sc_scatter(hp3, srcb, dstb)
    x3, hp4 = _tc_mid(acc, hp3, dinv, b3[None, :], W4, x2)
    acc = _sc_scatter(hp4, srcb, dstb)

    return _tc_final(acc, hp4, dinv, b4[None, :], x3,
                     batch[:, None], physchem,
                     ln_g[None, :], ln_b[None, :], pe_W1, pe_b1[None, :],
                     pe_W2, pe_b2[None, :], proj_W1, proj_b1[None, :],
                     proj_W2, proj_b2[None, :], gate_W, gate_b[None, :])
